# Initial kernel scaffold; baseline (speedup 1.0000x reference)
import jax
import jax.numpy as jnp
from jax.experimental import pallas as pl


def kernel(vertex_feat, edge_feat, edge_weight, incidence, inv_edge_degree, inv_vertex_degree, edge_scale, knn_k, conv_w, conv_b, bn_gamma, bn_beta):
    raise NotImplementedError("write your pallas kernel here")


# speedup vs baseline: 1.1193x; 1.1193x over previous
import functools

import jax
import jax.numpy as jnp
from jax.experimental import pallas as pl
from jax.experimental.pallas import tpu as pltpu

B, C, N, E = 4, 128, 2048, 2048
EB = 512
NB = 512
BN_EPS = 1e-5


def _pass1_body(vf_ref, inc_ref, ef_ref, ied_ref, w1_ref, w2_ref, b_ref,
                y_ref, stats_ref):
    b = pl.program_id(0)
    eb = pl.program_id(1)
    vf = vf_ref[0].astype(jnp.bfloat16)
    inc = inc_ref[0].astype(jnp.bfloat16)
    a = jnp.dot(vf, inc, preferred_element_type=jnp.float32)
    a = a * ied_ref[0]
    w1 = w1_ref[...].astype(jnp.bfloat16)
    w2 = w2_ref[...].astype(jnp.bfloat16)
    ef = ef_ref[0].astype(jnp.bfloat16)
    y = (jnp.dot(w1, ef, preferred_element_type=jnp.float32)
         + jnp.dot(w2, a.astype(jnp.bfloat16), preferred_element_type=jnp.float32)
         + b_ref[...])
    y_ref[0] = y
    s = jnp.sum(y, axis=1, keepdims=True)
    s2 = jnp.sum(y * y, axis=1, keepdims=True)
    stats = jnp.concatenate([s, s2], axis=1)

    @pl.when(jnp.logical_and(b == 0, eb == 0))
    def _init():
        stats_ref[...] = stats

    @pl.when(jnp.logical_not(jnp.logical_and(b == 0, eb == 0)))
    def _acc():
        stats_ref[...] += stats


def _pass2_body(y_ref, stats_ref, es_ref, g_ref, bt_ref, inc_ref, ivd_ref,
                vout_ref, eout_ref, zz_ref):
    nb = pl.program_id(1)

    @pl.when(nb == 0)
    def _normalize():
        cnt = float(B * E)
        mean = stats_ref[:, 0:1] / cnt
        var = stats_ref[:, 1:2] / cnt - mean * mean
        scale = g_ref[...] * jax.lax.rsqrt(var + BN_EPS)
        shift = bt_ref[...] - mean * scale
        z = y_ref[0] * scale + shift
        z = jnp.where(z >= 0, z, 0.2 * z)
        eout_ref[0] = z
        zz_ref[...] = (z * es_ref[0]).astype(jnp.bfloat16)

    inc = inc_ref[0].astype(jnp.bfloat16)
    v = jax.lax.dot_general(zz_ref[...], inc, (((1,), (1,)), ((), ())),
                            preferred_element_type=jnp.float32)
    vout_ref[0] = v * ivd_ref[0]


@jax.jit
def kernel(vertex_feat, edge_feat, edge_weight, incidence, inv_edge_degree,
           inv_vertex_degree, edge_scale, knn_k, conv_w, conv_b, bn_gamma,
           bn_beta):
    del edge_weight, knn_k
    w1 = conv_w[:, :C]
    w2 = conv_w[:, C:]
    bcol = conv_b[:, None]
    gcol = bn_gamma[:, None]
    btcol = bn_beta[:, None]
    ied_row = inv_edge_degree[:, :, 0][:, None, :]
    ivd_row = inv_vertex_degree[:, :, 0][:, None, :]
    es_row = edge_scale[:, :, 0][:, None, :]

    y, stats = pl.pallas_call(
        _pass1_body,
        grid=(B, E // EB),
        in_specs=[
            pl.BlockSpec((1, C, N), lambda b, e: (b, 0, 0)),
            pl.BlockSpec((1, N, EB), lambda b, e: (b, 0, e)),
            pl.BlockSpec((1, C, EB), lambda b, e: (b, 0, e)),
            pl.BlockSpec((1, 1, EB), lambda b, e: (b, 0, e)),
            pl.BlockSpec((C, C), lambda b, e: (0, 0)),
            pl.BlockSpec((C, C), lambda b, e: (0, 0)),
            pl.BlockSpec((C, 1), lambda b, e: (0, 0)),
        ],
        out_specs=[
            pl.BlockSpec((1, C, EB), lambda b, e: (b, 0, e)),
            pl.BlockSpec((C, 2), lambda b, e: (0, 0)),
        ],
        out_shape=[
            jax.ShapeDtypeStruct((B, C, E), jnp.float32),
            jax.ShapeDtypeStruct((C, 2), jnp.float32),
        ],
    )(vertex_feat, incidence, edge_feat, ied_row, w1, w2, bcol)

    vout, eout = pl.pallas_call(
        _pass2_body,
        grid=(B, N // NB),
        in_specs=[
            pl.BlockSpec((1, C, E), lambda b, n: (b, 0, 0)),
            pl.BlockSpec((C, 2), lambda b, n: (0, 0)),
            pl.BlockSpec((1, 1, E), lambda b, n: (b, 0, 0)),
            pl.BlockSpec((C, 1), lambda b, n: (0, 0)),
            pl.BlockSpec((C, 1), lambda b, n: (0, 0)),
            pl.BlockSpec((1, NB, E), lambda b, n: (b, n, 0)),
            pl.BlockSpec((1, 1, NB), lambda b, n: (b, 0, n)),
        ],
        out_specs=[
            pl.BlockSpec((1, C, NB), lambda b, n: (b, 0, n)),
            pl.BlockSpec((1, C, E), lambda b, n: (b, 0, 0)),
        ],
        out_shape=[
            jax.ShapeDtypeStruct((B, C, N), jnp.float32),
            jax.ShapeDtypeStruct((B, C, E), jnp.float32),
        ],
        scratch_shapes=[pltpu.VMEM((C, E), jnp.bfloat16)],
    )(y, stats, es_row, gcol, btcol, incidence, ivd_row)

    return (vout, eout)


# speedup vs baseline: 1.3407x; 1.1978x over previous
import jax
import jax.numpy as jnp
from jax.experimental import pallas as pl
from jax.experimental.pallas import tpu as pltpu

B, C, N, E = 4, 128, 2048, 2048
T = 4
EB = E // T
NB = N // T
BN_EPS = 1e-5


def _body(vf_ref, inc_ref, ef_ref, ied_ref, w1_ref, w2_ref, b_ref,
          es_ref, g_ref, bt_ref, ivd_ref,
          vout_ref, eout_ref,
          inc_cache, y_cache, zz_ref, stats_ref):
    ph = pl.program_id(0)
    b = pl.program_id(1)
    t = pl.program_id(2)

    @pl.when(ph == 0)
    def _phase0():
        inc_bf = inc_ref[0].astype(jnp.bfloat16)
        inc_cache[b, :, pl.ds(t * EB, EB)] = inc_bf
        vf = vf_ref[0].astype(jnp.bfloat16)
        a = jnp.dot(vf, inc_bf, preferred_element_type=jnp.float32)
        a = a * ied_ref[0]
        w1 = w1_ref[...].astype(jnp.bfloat16)
        w2 = w2_ref[...].astype(jnp.bfloat16)
        ef = ef_ref[0].astype(jnp.bfloat16)
        y = (jnp.dot(w1, ef, preferred_element_type=jnp.float32)
             + jnp.dot(w2, a.astype(jnp.bfloat16),
                       preferred_element_type=jnp.float32)
             + b_ref[...])
        y_cache[b, :, pl.ds(t * EB, EB)] = y
        s = jnp.sum(y, axis=1, keepdims=True)
        s2 = jnp.sum(y * y, axis=1, keepdims=True)
        stats = jnp.concatenate([s, s2], axis=1)

        @pl.when(jnp.logical_and(b == 0, t == 0))
        def _init():
            stats_ref[...] = stats

        @pl.when(jnp.logical_not(jnp.logical_and(b == 0, t == 0)))
        def _acc():
            stats_ref[...] += stats

    @pl.when(ph == 1)
    def _phase1():
        @pl.when(t == 0)
        def _normalize():
            cnt = float(B * E)
            mean = stats_ref[:, 0:1] / cnt
            var = stats_ref[:, 1:2] / cnt - mean * mean
            scale = g_ref[...] * jax.lax.rsqrt(var + BN_EPS)
            shift = bt_ref[...] - mean * scale
            z = y_cache[b] * scale + shift
            z = jnp.where(z >= 0, z, 0.2 * z)
            eout_ref[0] = z
            zz_ref[...] = (z * es_ref[0]).astype(jnp.bfloat16)

        inc_t = inc_cache[b, pl.ds(t * NB, NB), :]
        v = jax.lax.dot_general(zz_ref[...], inc_t, (((1,), (1,)), ((), ())),
                                preferred_element_type=jnp.float32)
        vout_ref[0] = v * ivd_ref[0]


@jax.jit
def kernel(vertex_feat, edge_feat, edge_weight, incidence, inv_edge_degree,
           inv_vertex_degree, edge_scale, knn_k, conv_w, conv_b, bn_gamma,
           bn_beta):
    del edge_weight, knn_k
    w1 = conv_w[:, :C]
    w2 = conv_w[:, C:]
    bcol = conv_b[:, None]
    gcol = bn_gamma[:, None]
    btcol = bn_beta[:, None]
    ied_row = inv_edge_degree[:, :, 0][:, None, :]
    ivd_row = inv_vertex_degree[:, :, 0][:, None, :]
    es_row = edge_scale[:, :, 0][:, None, :]

    vout, eout = pl.pallas_call(
        _body,
        grid=(2, B, T),
        in_specs=[
            pl.BlockSpec((1, C, N), lambda p, b, t: ((1 - p) * b, 0, 0)),
            pl.BlockSpec((1, N, EB),
                         lambda p, b, t: ((1 - p) * b, 0, (1 - p) * t)),
            pl.BlockSpec((1, C, EB),
                         lambda p, b, t: ((1 - p) * b, 0, (1 - p) * t)),
            pl.BlockSpec((1, 1, EB),
                         lambda p, b, t: ((1 - p) * b, 0, (1 - p) * t)),
            pl.BlockSpec((C, C), lambda p, b, t: (0, 0)),
            pl.BlockSpec((C, C), lambda p, b, t: (0, 0)),
            pl.BlockSpec((C, 1), lambda p, b, t: (0, 0)),
            pl.BlockSpec((1, 1, E), lambda p, b, t: (p * b, 0, 0)),
            pl.BlockSpec((C, 1), lambda p, b, t: (0, 0)),
            pl.BlockSpec((C, 1), lambda p, b, t: (0, 0)),
            pl.BlockSpec((1, 1, NB), lambda p, b, t: (p * b, 0, p * t)),
        ],
        out_specs=[
            pl.BlockSpec((1, C, NB), lambda p, b, t: (p * b, 0, p * t)),
            pl.BlockSpec((1, C, E), lambda p, b, t: (p * b, 0, 0)),
        ],
        out_shape=[
            jax.ShapeDtypeStruct((B, C, N), jnp.float32),
            jax.ShapeDtypeStruct((B, C, E), jnp.float32),
        ],
        scratch_shapes=[
            pltpu.VMEM((B, N, E), jnp.bfloat16),
            pltpu.VMEM((B, C, E), jnp.float32),
            pltpu.VMEM((C, E), jnp.bfloat16),
            pltpu.VMEM((C, 2), jnp.float32),
        ],
    )(vertex_feat, incidence, edge_feat, ied_row, w1, w2, bcol,
      es_row, gcol, btcol, ivd_row)

    return (vout, eout)


# speedup vs baseline: 1.3633x; 1.0168x over previous
import jax
import jax.numpy as jnp
from jax.experimental import pallas as pl
from jax.experimental.pallas import tpu as pltpu

B, C, N, E = 4, 128, 2048, 2048
T = 4
NB = N // T
BN_EPS = 1e-5


def _body(vf_ref, inc_ref, ef_ref, ied_ref, w1_ref, w2_ref, b_ref,
          es_ref, g_ref, bt_ref, ivd_ref,
          vout_ref, eout_ref,
          inc_cache, y_cache, a_acc, zz_ref, stats_ref):
    ph = pl.program_id(0)
    b = pl.program_id(1)
    t = pl.program_id(2)

    @pl.when(ph == 0)
    def _phase0():
        inc_bf = inc_ref[0].astype(jnp.bfloat16)
        inc_cache[b, pl.ds(t * NB, NB), :] = inc_bf
        vf_t = vf_ref[0, :, pl.ds(t * NB, NB)].astype(jnp.bfloat16)
        ap = jnp.dot(vf_t, inc_bf, preferred_element_type=jnp.float32)

        @pl.when(t == 0)
        def _first():
            a_acc[...] = ap

        @pl.when(t != 0)
        def _rest():
            a_acc[...] += ap

        @pl.when(t == T - 1)
        def _finish():
            a = (a_acc[...] * ied_ref[0]).astype(jnp.bfloat16)
            w1 = w1_ref[...].astype(jnp.bfloat16)
            w2 = w2_ref[...].astype(jnp.bfloat16)
            ef = ef_ref[0].astype(jnp.bfloat16)
            y = (jnp.dot(w1, ef, preferred_element_type=jnp.float32)
                 + jnp.dot(w2, a, preferred_element_type=jnp.float32)
                 + b_ref[...])
            y_cache[b] = y
            s = jnp.sum(y, axis=1, keepdims=True)
            s2 = jnp.sum(y * y, axis=1, keepdims=True)
            stats = jnp.concatenate([s, s2], axis=1)

            @pl.when(b == 0)
            def _init():
                stats_ref[...] = stats

            @pl.when(b != 0)
            def _acc():
                stats_ref[...] += stats

    @pl.when(ph == 1)
    def _phase1():
        @pl.when(t == 0)
        def _normalize():
            cnt = float(B * E)
            mean = stats_ref[:, 0:1] / cnt
            var = stats_ref[:, 1:2] / cnt - mean * mean
            scale = g_ref[...] * jax.lax.rsqrt(var + BN_EPS)
            shift = bt_ref[...] - mean * scale
            z = y_cache[b] * scale + shift
            z = jnp.where(z >= 0, z, 0.2 * z)
            eout_ref[0] = z
            zz_ref[...] = (z * es_ref[0]).astype(jnp.bfloat16)

        inc_t = inc_cache[b, pl.ds(t * NB, NB), :]
        v = jax.lax.dot_general(zz_ref[...], inc_t, (((1,), (1,)), ((), ())),
                                preferred_element_type=jnp.float32)
        vout_ref[0] = v * ivd_ref[0]


@jax.jit
def kernel(vertex_feat, edge_feat, edge_weight, incidence, inv_edge_degree,
           inv_vertex_degree, edge_scale, knn_k, conv_w, conv_b, bn_gamma,
           bn_beta):
    del edge_weight, knn_k
    w1 = conv_w[:, :C]
    w2 = conv_w[:, C:]
    bcol = conv_b[:, None]
    gcol = bn_gamma[:, None]
    btcol = bn_beta[:, None]
    ied_row = inv_edge_degree[:, :, 0][:, None, :]
    ivd_row = inv_vertex_degree[:, :, 0][:, None, :]
    es_row = edge_scale[:, :, 0][:, None, :]

    vout, eout = pl.pallas_call(
        _body,
        grid=(2, B, T),
        in_specs=[
            pl.BlockSpec((1, C, N), lambda p, b, t: ((1 - p) * b, 0, 0)),
            pl.BlockSpec((1, NB, E),
                         lambda p, b, t: ((1 - p) * b, (1 - p) * t, 0)),
            pl.BlockSpec((1, C, E), lambda p, b, t: ((1 - p) * b, 0, 0)),
            pl.BlockSpec((1, 1, E), lambda p, b, t: ((1 - p) * b, 0, 0)),
            pl.BlockSpec((C, C), lambda p, b, t: (0, 0)),
            pl.BlockSpec((C, C), lambda p, b, t: (0, 0)),
            pl.BlockSpec((C, 1), lambda p, b, t: (0, 0)),
            pl.BlockSpec((1, 1, E), lambda p, b, t: (p * b, 0, 0)),
            pl.BlockSpec((C, 1), lambda p, b, t: (0, 0)),
            pl.BlockSpec((C, 1), lambda p, b, t: (0, 0)),
            pl.BlockSpec((1, 1, NB), lambda p, b, t: (p * b, 0, p * t)),
        ],
        out_specs=[
            pl.BlockSpec((1, C, NB), lambda p, b, t: (p * b, 0, p * t)),
            pl.BlockSpec((1, C, E), lambda p, b, t: (p * b, 0, 0)),
        ],
        out_shape=[
            jax.ShapeDtypeStruct((B, C, N), jnp.float32),
            jax.ShapeDtypeStruct((B, C, E), jnp.float32),
        ],
        scratch_shapes=[
            pltpu.VMEM((B, N, E), jnp.bfloat16),
            pltpu.VMEM((B, C, E), jnp.float32),
            pltpu.VMEM((C, E), jnp.float32),
            pltpu.VMEM((C, E), jnp.bfloat16),
            pltpu.VMEM((C, 2), jnp.float32),
        ],
    )(vertex_feat, incidence, edge_feat, ied_row, w1, w2, bcol,
      es_row, gcol, btcol, ivd_row)

    return (vout, eout)


# speedup vs baseline: 1.3725x; 1.0067x over previous
import jax
import jax.numpy as jnp
from jax.experimental import pallas as pl
from jax.experimental.pallas import tpu as pltpu

B, C, N, E = 4, 128, 2048, 2048
T = 4
NB = N // T
BN_EPS = 1e-5


def _body(vf_ref, inc_lo_ref, inc_hi_ref, ef_ref, ied_ref, w1_ref, w2_ref,
          b_ref, es_ref, g_ref, bt_ref, ivd_ref,
          vout_ref, eout_ref,
          inc_cache, y_cache, a_acc, zz_ref, stats_ref):
    ph = pl.program_id(0)
    b = pl.program_id(1)
    t = pl.program_id(2)
    E2 = E // 2

    @pl.when(ph == 0)
    def _phase0():
        inc_lo = inc_lo_ref[0].astype(jnp.bfloat16)
        inc_hi = inc_hi_ref[0].astype(jnp.bfloat16)
        inc_cache[b, pl.ds(t * NB, NB), :E2] = inc_lo
        inc_cache[b, pl.ds(t * NB, NB), E2:] = inc_hi
        vf_t = vf_ref[0, :, pl.ds(t * NB, NB)].astype(jnp.bfloat16)
        ap_lo = jnp.dot(vf_t, inc_lo, preferred_element_type=jnp.float32)
        ap_hi = jnp.dot(vf_t, inc_hi, preferred_element_type=jnp.float32)

        @pl.when(t == 0)
        def _first():
            a_acc[:, :E2] = ap_lo
            a_acc[:, E2:] = ap_hi

        @pl.when(t != 0)
        def _rest():
            a_acc[:, :E2] += ap_lo
            a_acc[:, E2:] += ap_hi

        @pl.when(t == T - 1)
        def _finish():
            a = (a_acc[...] * ied_ref[0]).astype(jnp.bfloat16)
            w1 = w1_ref[...].astype(jnp.bfloat16)
            w2 = w2_ref[...].astype(jnp.bfloat16)
            ef = ef_ref[0].astype(jnp.bfloat16)
            y = (jnp.dot(w1, ef, preferred_element_type=jnp.float32)
                 + jnp.dot(w2, a, preferred_element_type=jnp.float32)
                 + b_ref[...])
            y_cache[b] = y
            s = jnp.sum(y, axis=1, keepdims=True)
            s2 = jnp.sum(y * y, axis=1, keepdims=True)
            stats = jnp.concatenate([s, s2], axis=1)

            @pl.when(b == 0)
            def _init():
                stats_ref[...] = stats

            @pl.when(b != 0)
            def _acc():
                stats_ref[...] += stats

    @pl.when(ph == 1)
    def _phase1():
        @pl.when(t == 0)
        def _normalize():
            cnt = float(B * E)
            mean = stats_ref[:, 0:1] / cnt
            var = stats_ref[:, 1:2] / cnt - mean * mean
            scale = g_ref[...] * jax.lax.rsqrt(var + BN_EPS)
            shift = bt_ref[...] - mean * scale
            z = y_cache[b] * scale + shift
            z = jnp.where(z >= 0, z, 0.2 * z)
            eout_ref[0] = z
            zz_ref[...] = (z * es_ref[0]).astype(jnp.bfloat16)

        inc_t = inc_cache[b, pl.ds(t * NB, NB), :]
        v = jax.lax.dot_general(zz_ref[...], inc_t, (((1,), (1,)), ((), ())),
                                preferred_element_type=jnp.float32)
        vout_ref[0] = v * ivd_ref[0]


@jax.jit
def kernel(vertex_feat, edge_feat, edge_weight, incidence, inv_edge_degree,
           inv_vertex_degree, edge_scale, knn_k, conv_w, conv_b, bn_gamma,
           bn_beta):
    del edge_weight, knn_k
    w1 = conv_w[:, :C]
    w2 = conv_w[:, C:]
    bcol = conv_b[:, None]
    gcol = bn_gamma[:, None]
    btcol = bn_beta[:, None]
    ied_row = inv_edge_degree[:, :, 0][:, None, :]
    ivd_row = inv_vertex_degree[:, :, 0][:, None, :]
    es_row = edge_scale[:, :, 0][:, None, :]

    vout, eout = pl.pallas_call(
        _body,
        grid=(2, B, T),
        in_specs=[
            pl.BlockSpec((1, C, N), lambda p, b, t: ((1 - p) * b, 0, 0)),
            pl.BlockSpec((1, NB, E // 2),
                         lambda p, b, t: ((1 - p) * b, (1 - p) * t, 0)),
            pl.BlockSpec((1, NB, E // 2),
                         lambda p, b, t: ((1 - p) * b, (1 - p) * t, 1)),
            pl.BlockSpec((1, C, E), lambda p, b, t: ((1 - p) * b, 0, 0)),
            pl.BlockSpec((1, 1, E), lambda p, b, t: ((1 - p) * b, 0, 0)),
            pl.BlockSpec((C, C), lambda p, b, t: (0, 0)),
            pl.BlockSpec((C, C), lambda p, b, t: (0, 0)),
            pl.BlockSpec((C, 1), lambda p, b, t: (0, 0)),
            pl.BlockSpec((1, 1, E), lambda p, b, t: (p * b, 0, 0)),
            pl.BlockSpec((C, 1), lambda p, b, t: (0, 0)),
            pl.BlockSpec((C, 1), lambda p, b, t: (0, 0)),
            pl.BlockSpec((1, 1, NB), lambda p, b, t: (p * b, 0, p * t)),
        ],
        out_specs=[
            pl.BlockSpec((1, C, NB), lambda p, b, t: (p * b, 0, p * t)),
            pl.BlockSpec((1, C, E), lambda p, b, t: (p * b, 0, 0)),
        ],
        out_shape=[
            jax.ShapeDtypeStruct((B, C, N), jnp.float32),
            jax.ShapeDtypeStruct((B, C, E), jnp.float32),
        ],
        scratch_shapes=[
            pltpu.VMEM((B, N, E), jnp.bfloat16),
            pltpu.VMEM((B, C, E), jnp.float32),
            pltpu.VMEM((C, E), jnp.float32),
            pltpu.VMEM((C, E), jnp.bfloat16),
            pltpu.VMEM((C, 2), jnp.float32),
        ],
    )(vertex_feat, incidence, incidence, edge_feat, ied_row, w1, w2, bcol,
      es_row, gcol, btcol, ivd_row)

    return (vout, eout)


# speedup vs baseline: 1.5363x; 1.1194x over previous
import jax
import jax.numpy as jnp
from jax.experimental import pallas as pl
from jax.experimental.pallas import tpu as pltpu

B, C, N, E = 4, 128, 2048, 2048
T = 4
NB = N // T
P0 = B * T
BN_EPS = 1e-5


def _body(vf_ref, inc_ref, ef_ref, rows_ref, par_ref,
          vout_ref, eout_ref,
          inc_cache, y_cache, a_acc, stats_ref):
    s = pl.program_id(0)

    @pl.when(s < P0)
    def _phase0():
        b = s // T
        t = s % T
        inc_bf = inc_ref[0].astype(jnp.bfloat16)
        inc_cache[b, pl.ds(t * NB, NB), :] = inc_bf
        vf_t = vf_ref[0, :, pl.ds(t * NB, NB)].astype(jnp.bfloat16)
        ap = jnp.dot(vf_t, inc_bf, preferred_element_type=jnp.float32)

        @pl.when(t == 0)
        def _first():
            a_acc[...] = ap

        @pl.when(t != 0)
        def _rest():
            a_acc[...] += ap

        @pl.when(t == T - 1)
        def _finish():
            ied = rows_ref[0, 0:1, :]
            a = (a_acc[...] * ied).astype(jnp.bfloat16)
            w1 = par_ref[:, 0:C].astype(jnp.bfloat16)
            w2 = par_ref[:, C:2 * C].astype(jnp.bfloat16)
            bcol = par_ref[:, 2 * C:2 * C + 1]
            ef = ef_ref[0].astype(jnp.bfloat16)
            y = (jnp.dot(w1, ef, preferred_element_type=jnp.float32)
                 + jnp.dot(w2, a, preferred_element_type=jnp.float32)
                 + bcol)
            y_cache[b] = y
            st = jnp.concatenate(
                [jnp.sum(y, axis=1, keepdims=True),
                 jnp.sum(y * y, axis=1, keepdims=True)], axis=1)

            @pl.when(b == 0)
            def _init():
                stats_ref[...] = st

            @pl.when(b != 0)
            def _acc():
                stats_ref[...] += st

    @pl.when(s >= P0)
    def _phase1():
        b = s - P0
        cnt = float(B * E)
        mean = stats_ref[:, 0:1] / cnt
        var = stats_ref[:, 1:2] / cnt - mean * mean
        scale = par_ref[:, 2 * C + 1:2 * C + 2] * jax.lax.rsqrt(var + BN_EPS)
        shift = par_ref[:, 2 * C + 2:2 * C + 3] - mean * scale
        z = y_cache[b] * scale + shift
        z = jnp.where(z >= 0, z, 0.2 * z)
        eout_ref[0] = z
        es = rows_ref[0, 1:2, :]
        zz = (z * es).astype(jnp.bfloat16)
        inc_b = inc_cache[b]
        v = jax.lax.dot_general(zz, inc_b, (((1,), (1,)), ((), ())),
                                preferred_element_type=jnp.float32)
        ivd = rows_ref[0, 2:3, :]
        vout_ref[0] = v * ivd


@jax.jit
def kernel(vertex_feat, edge_feat, edge_weight, incidence, inv_edge_degree,
           inv_vertex_degree, edge_scale, knn_k, conv_w, conv_b, bn_gamma,
           bn_beta):
    del edge_weight, knn_k
    par = jnp.concatenate(
        [conv_w, conv_b[:, None], bn_gamma[:, None], bn_beta[:, None]],
        axis=1)
    rows = jnp.concatenate(
        [inv_edge_degree[:, :, 0][:, None, :],
         edge_scale[:, :, 0][:, None, :],
         inv_vertex_degree[:, :, 0][:, None, :]], axis=1)

    def b0(s):
        return jnp.where(s < P0, s // T, 0)

    def b_any(s):
        return jnp.where(s < P0, s // T, s - P0)

    def b1(s):
        return jnp.where(s < P0, 0, s - P0)

    vout, eout = pl.pallas_call(
        _body,
        grid=(P0 + B,),
        in_specs=[
            pl.BlockSpec((1, C, N), lambda s: (b0(s), 0, 0)),
            pl.BlockSpec((1, NB, E),
                         lambda s: (b0(s), jnp.where(s < P0, s % T, 0), 0)),
            pl.BlockSpec((1, C, E), lambda s: (b0(s), 0, 0)),
            pl.BlockSpec((1, 3, E), lambda s: (b_any(s), 0, 0)),
            pl.BlockSpec((C, 2 * C + 3), lambda s: (0, 0)),
        ],
        out_specs=[
            pl.BlockSpec((1, C, N), lambda s: (b1(s), 0, 0)),
            pl.BlockSpec((1, C, E), lambda s: (b1(s), 0, 0)),
        ],
        out_shape=[
            jax.ShapeDtypeStruct((B, C, N), jnp.float32),
            jax.ShapeDtypeStruct((B, C, E), jnp.float32),
        ],
        scratch_shapes=[
            pltpu.VMEM((B, N, E), jnp.bfloat16),
            pltpu.VMEM((B, C, E), jnp.float32),
            pltpu.VMEM((C, E), jnp.float32),
            pltpu.VMEM((C, 2), jnp.float32),
        ],
    )(vertex_feat, incidence, edge_feat, rows, par)

    return (vout, eout)


# speedup vs baseline: 1.6233x; 1.0566x over previous
import jax
import jax.numpy as jnp
from jax.experimental import pallas as pl
from jax.experimental.pallas import tpu as pltpu

B, C, N, E = 4, 128, 2048, 2048
T = 2
NB = N // T
P0 = B * T
BN_EPS = 1e-5


def _body(vf_ref, inc_ref, ef_ref, rows_ref, par_ref,
          vout_ref, eout_ref,
          inc_cache, y_cache, a_acc, stats_ref):
    s = pl.program_id(0)

    @pl.when(s < P0)
    def _phase0():
        b = s // T
        t = s % T
        inc_bf = inc_ref[0].astype(jnp.bfloat16)
        inc_cache[b, pl.ds(t * NB, NB), :] = inc_bf
        vf_t = vf_ref[0].astype(jnp.bfloat16)
        ap = jnp.dot(vf_t, inc_bf, preferred_element_type=jnp.float32)

        @pl.when(t == 0)
        def _first():
            a_acc[...] = ap

        @pl.when(t != 0)
        def _rest():
            a_acc[...] += ap

        @pl.when(t == T - 1)
        def _finish():
            ied = rows_ref[0, 0:1, :]
            a = (a_acc[...] * ied).astype(jnp.bfloat16)
            w1 = par_ref[:, 0:C].astype(jnp.bfloat16)
            w2 = par_ref[:, C:2 * C].astype(jnp.bfloat16)
            bcol = par_ref[:, 2 * C:2 * C + 1]
            ef = ef_ref[0].astype(jnp.bfloat16)
            y = (jnp.dot(w1, ef, preferred_element_type=jnp.float32)
                 + jnp.dot(w2, a, preferred_element_type=jnp.float32)
                 + bcol)
            y_cache[b] = y.astype(jnp.bfloat16)
            st = jnp.concatenate(
                [jnp.sum(y, axis=1, keepdims=True),
                 jnp.sum(y * y, axis=1, keepdims=True)], axis=1)

            @pl.when(b == 0)
            def _init():
                stats_ref[...] = st

            @pl.when(b != 0)
            def _acc():
                stats_ref[...] += st

    @pl.when(s >= P0)
    def _phase1():
        b = s - P0
        cnt = float(B * E)
        mean = stats_ref[:, 0:1] / cnt
        var = stats_ref[:, 1:2] / cnt - mean * mean
        scale = par_ref[:, 2 * C + 1:2 * C + 2] * jax.lax.rsqrt(var + BN_EPS)
        shift = par_ref[:, 2 * C + 2:2 * C + 3] - mean * scale
        z = y_cache[b].astype(jnp.float32) * scale + shift
        z = jnp.where(z >= 0, z, 0.2 * z)
        eout_ref[0] = z
        es = rows_ref[0, 1:2, :]
        zz = (z * es).astype(jnp.bfloat16)
        inc_b = inc_cache[b]
        v = jax.lax.dot_general(zz, inc_b, (((1,), (1,)), ((), ())),
                                preferred_element_type=jnp.float32)
        ivd = rows_ref[0, 2:3, :]
        vout_ref[0] = v * ivd


@jax.jit
def kernel(vertex_feat, edge_feat, edge_weight, incidence, inv_edge_degree,
           inv_vertex_degree, edge_scale, knn_k, conv_w, conv_b, bn_gamma,
           bn_beta):
    del edge_weight, knn_k
    par = jnp.concatenate(
        [conv_w, conv_b[:, None], bn_gamma[:, None], bn_beta[:, None]],
        axis=1)
    rows = jnp.concatenate(
        [inv_edge_degree[:, :, 0][:, None, :],
         edge_scale[:, :, 0][:, None, :],
         inv_vertex_degree[:, :, 0][:, None, :]], axis=1)

    def b0(s):
        return jnp.where(s < P0, s // T, 0)

    def b_any(s):
        return jnp.where(s < P0, s // T, s - P0)

    def b1(s):
        return jnp.where(s < P0, 0, s - P0)

    vout, eout = pl.pallas_call(
        _body,
        grid=(P0 + B,),
        in_specs=[
            pl.BlockSpec((1, C, NB),
                         lambda s: (b0(s), 0, jnp.where(s < P0, s % T, 0))),
            pl.BlockSpec((1, NB, E),
                         lambda s: (b0(s), jnp.where(s < P0, s % T, 0), 0)),
            pl.BlockSpec((1, C, E), lambda s: (b0(s), 0, 0)),
            pl.BlockSpec((1, 3, E), lambda s: (b_any(s), 0, 0)),
            pl.BlockSpec((C, 2 * C + 3), lambda s: (0, 0)),
        ],
        out_specs=[
            pl.BlockSpec((1, C, N), lambda s: (b1(s), 0, 0)),
            pl.BlockSpec((1, C, E), lambda s: (b1(s), 0, 0)),
        ],
        out_shape=[
            jax.ShapeDtypeStruct((B, C, N), jnp.float32),
            jax.ShapeDtypeStruct((B, C, E), jnp.float32),
        ],
        scratch_shapes=[
            pltpu.VMEM((B, N, E), jnp.bfloat16),
            pltpu.VMEM((B, C, E), jnp.bfloat16),
            pltpu.VMEM((C, E), jnp.float32),
            pltpu.VMEM((C, 2), jnp.float32),
        ],
        compiler_params=pltpu.CompilerParams(
            vmem_limit_bytes=63 * 1024 * 1024),
    )(vertex_feat, incidence, edge_feat, rows, par)

    return (vout, eout)
